# baseline (speedup 1.0000x reference)
import functools

import jax
import jax.numpy as jnp
from jax import lax
from jax.experimental import pallas as pl
from jax.experimental.pallas import tpu as pltpu
from jax.experimental.pallas import tpu_sc as plsc

DIM = 32
BATCH = 16384
PACK = 128 // DIM

_INFO = plsc.get_sparse_core_info()
_NC, _NS, _L = _INFO.num_cores, _INFO.num_subcores, _INFO.num_lanes
_NW = _NC * _NS
_BPW = BATCH // _NW
_CHUNK = 128
_NCHUNK = _BPW // _CHUNK

_TROWS = 512


def _tpose_body(x_ref, out_ref):
    eye = jnp.eye(DIM, dtype=jnp.float32)
    xt = lax.dot_general(x_ref[...], eye, (((0,), (0,)), ((), ())),
                         preferred_element_type=jnp.float32)
    out_ref[...] = jnp.concatenate(
        [xt[q * _TROWS:(q + 1) * _TROWS] for q in range(PACK)], axis=1)


def _transpose_pack(gT, q_pad):
    dim, n = gT.shape
    grid = (q_pad // _TROWS,)
    return pl.pallas_call(
        _tpose_body,
        grid=grid,
        in_specs=[pl.BlockSpec((dim, PACK * _TROWS), lambda i: (0, i))],
        out_specs=pl.BlockSpec((_TROWS, PACK * dim), lambda i: (i, 0)),
        out_shape=jax.ShapeDtypeStruct((q_pad, PACK * dim), jnp.float32),
    )(gT)


_mesh = plsc.VectorSubcoreMesh(core_axis_name="c", subcore_axis_name="s")


@functools.partial(
    pl.kernel,
    mesh=_mesh,
    out_type=jax.ShapeDtypeStruct((BATCH,), jnp.float32),
    compiler_params=pltpu.CompilerParams(
        needs_layout_passes=False, use_tc_tiling_on_sc=True),
    scratch_types=[
        pltpu.VMEM((_NCHUNK, _CHUNK), jnp.int32),
        pltpu.VMEM((_NCHUNK, _CHUNK), jnp.int32),
        pltpu.VMEM((_NCHUNK, _CHUNK), jnp.int32),
        pltpu.VMEM((_NCHUNK, _CHUNK), jnp.int32),
        pltpu.VMEM((_NCHUNK, _CHUNK), jnp.int32),
        pltpu.VMEM((_NCHUNK, _CHUNK), jnp.int32),
        pltpu.VMEM((2, _CHUNK, 128), jnp.float32),
        pltpu.VMEM((2, _CHUNK, 128), jnp.float32),
        pltpu.VMEM((2, _CHUNK, 128), jnp.float32),
        pltpu.VMEM((_BPW,), jnp.float32),
        pltpu.SemaphoreType.DMA,
        pltpu.SemaphoreType.DMA,
    ],
)
def _bpr_sc(ui_hbm, pi_hbm, ni_hbm, gu_hbm, gi_hbm, out_hbm,
            ui_v, pi_v, ni_v, upk_v, ppk_v, npk_v,
            u_rows, p_rows, n_rows, out_v, sem0, sem1):
    wid = lax.axis_index("s") * _NC + lax.axis_index("c")
    base = wid * _BPW
    sems = (sem0, sem1)

    pltpu.sync_copy(ui_hbm.at[wid], ui_v)
    pltpu.sync_copy(pi_hbm.at[wid], pi_v)
    pltpu.sync_copy(ni_hbm.at[wid], ni_v)

    def _pk(v):
        return lax.shift_left(lax.shift_right_logical(v, 11), 9) | (v & 511)

    for c in range(_NCHUNK):
        for g in range(_CHUNK // _L):
            s = pl.ds(g * _L, _L)
            upk_v[c, s] = _pk(ui_v[c, s])
            ppk_v[c, s] = _pk(pi_v[c, s])
            npk_v[c, s] = _pk(ni_v[c, s])

    def fire(c):
        b = c % 2
        sem = sems[b]
        return (pltpu.async_copy(gu_hbm.at[upk_v.at[c]], u_rows.at[b], sem),
                pltpu.async_copy(gi_hbm.at[ppk_v.at[c]], p_rows.at[b], sem),
                pltpu.async_copy(gi_hbm.at[npk_v.at[c]], n_rows.at[b], sem))

    lanes = lax.iota(jnp.int32, _L)
    inflight = fire(0)

    for c in range(_NCHUNK):
        b = c % 2
        cur = inflight
        if c + 1 < _NCHUNK:
            inflight = fire(c + 1)
        for cp in cur:
            cp.wait()

        ub, pb, nb = u_rows.at[b], p_rows.at[b], n_rows.at[b]

        def group(g, _, c=c, ub=ub, pb=pb, nb=nb):
            s = pl.ds(g * _L, _L)
            colu = (lax.shift_right_logical(ui_v[c, s], 9) & 3) * DIM
            colp = (lax.shift_right_logical(pi_v[c, s], 9) & 3) * DIM
            coln = (lax.shift_right_logical(ni_v[c, s], 9) & 3) * DIM
            rowv = g * _L + lanes
            acc = jnp.zeros((_L,), jnp.float32)
            for d in range(DIM):
                u = plsc.load_gather(ub, [rowv, colu + d])
                p = plsc.load_gather(pb, [rowv, colp + d])
                n = plsc.load_gather(nb, [rowv, coln + d])
                acc = acc + u * (p - n)
            out_v[pl.ds(c * _CHUNK + g * _L, _L)] = acc
            return 0

        lax.fori_loop(0, _CHUNK // _L, group, 0)

    pltpu.sync_copy(out_v, out_hbm.at[pl.ds(base, _BPW)])


def kernel(ui, pi, ni, gamma_users, gamma_items):
    n_users = gamma_users.shape[0]
    q_pad = -(-(n_users // PACK) // _TROWS) * _TROWS
    gu2 = _transpose_pack(gamma_users.T, q_pad)
    gi2 = _transpose_pack(gamma_items.T, q_pad)
    ui3 = ui.astype(jnp.int32).reshape(_NW, _NCHUNK, _CHUNK)
    pi3 = pi.astype(jnp.int32).reshape(_NW, _NCHUNK, _CHUNK)
    ni3 = ni.astype(jnp.int32).reshape(_NW, _NCHUNK, _CHUNK)
    return _bpr_sc(ui3, pi3, ni3, gu2, gi2)


# speedup vs baseline: 1.5962x; 1.5962x over previous
import functools

import jax
import jax.numpy as jnp
from jax import lax
from jax.experimental import pallas as pl
from jax.experimental.pallas import tpu as pltpu
from jax.experimental.pallas import tpu_sc as plsc

DIM = 32
BATCH = 16384
PACK = 128 // DIM

_INFO = plsc.get_sparse_core_info()
_NC, _NS, _L = _INFO.num_cores, _INFO.num_subcores, _INFO.num_lanes
_NW = _NC * _NS
_BPW = BATCH // _NW
_CHUNK = 128
_NCHUNK = _BPW // _CHUNK

_TROWS = 2048


def _tpose_body(x_ref, out_ref):
    eye = jnp.eye(DIM, dtype=jnp.float32)
    xt = lax.dot_general(x_ref[...], eye, (((0,), (0,)), ((), ())),
                         preferred_element_type=jnp.float32)
    out_ref[...] = jnp.concatenate(
        [xt[q * _TROWS:(q + 1) * _TROWS] for q in range(PACK)], axis=1)


def _transpose_pack(gT, q_pad):
    dim, n = gT.shape
    grid = (q_pad // _TROWS,)
    return pl.pallas_call(
        _tpose_body,
        grid=grid,
        in_specs=[pl.BlockSpec((dim, PACK * _TROWS), lambda i: (0, i))],
        out_specs=pl.BlockSpec((_TROWS, PACK * dim), lambda i: (i, 0)),
        out_shape=jax.ShapeDtypeStruct((q_pad, PACK * dim), jnp.float32),
    )(gT)


_mesh = plsc.VectorSubcoreMesh(core_axis_name="c", subcore_axis_name="s")


@functools.partial(
    pl.kernel,
    mesh=_mesh,
    out_type=jax.ShapeDtypeStruct((BATCH,), jnp.float32),
    compiler_params=pltpu.CompilerParams(
        needs_layout_passes=False, use_tc_tiling_on_sc=True),
    scratch_types=[
        pltpu.VMEM((_NCHUNK, _CHUNK), jnp.int32),
        pltpu.VMEM((_NCHUNK, _CHUNK), jnp.int32),
        pltpu.VMEM((_NCHUNK, _CHUNK), jnp.int32),
        pltpu.VMEM((_NCHUNK, _CHUNK), jnp.int32),
        pltpu.VMEM((_NCHUNK, _CHUNK), jnp.int32),
        pltpu.VMEM((_NCHUNK, _CHUNK), jnp.int32),
        pltpu.VMEM((2, _CHUNK, 128), jnp.float32),
        pltpu.VMEM((2, _CHUNK, 128), jnp.float32),
        pltpu.VMEM((2, _CHUNK, 128), jnp.float32),
        pltpu.VMEM((_BPW,), jnp.float32),
        pltpu.SemaphoreType.DMA,
        pltpu.SemaphoreType.DMA,
    ],
)
def _bpr_sc(ui_hbm, pi_hbm, ni_hbm, gu_hbm, gi_hbm, out_hbm,
            ui_v, pi_v, ni_v, upk_v, ppk_v, npk_v,
            u_rows, p_rows, n_rows, out_v, sem0, sem1):
    wid = lax.axis_index("s") * _NC + lax.axis_index("c")
    base = wid * _BPW
    sems = (sem0, sem1)

    pltpu.sync_copy(ui_hbm.at[wid], ui_v)
    pltpu.sync_copy(pi_hbm.at[wid], pi_v)
    pltpu.sync_copy(ni_hbm.at[wid], ni_v)

    def _pk(v):
        return lax.shift_left(lax.shift_right_logical(v, 13), 11) | (v & 2047)

    for c in range(_NCHUNK):
        for g in range(_CHUNK // _L):
            s = pl.ds(g * _L, _L)
            upk_v[c, s] = _pk(ui_v[c, s])
            ppk_v[c, s] = _pk(pi_v[c, s])
            npk_v[c, s] = _pk(ni_v[c, s])

    def fire(c):
        b = c % 2
        sem = sems[b]
        return (pltpu.async_copy(gu_hbm.at[upk_v.at[c]], u_rows.at[b], sem),
                pltpu.async_copy(gi_hbm.at[ppk_v.at[c]], p_rows.at[b], sem),
                pltpu.async_copy(gi_hbm.at[npk_v.at[c]], n_rows.at[b], sem))

    lanes = lax.iota(jnp.int32, _L)
    inflight = fire(0)

    for c in range(_NCHUNK):
        b = c % 2
        cur = inflight
        if c + 1 < _NCHUNK:
            inflight = fire(c + 1)
        for cp in cur:
            cp.wait()

        ub, pb, nb = u_rows.at[b], p_rows.at[b], n_rows.at[b]

        def group(g, _, c=c, ub=ub, pb=pb, nb=nb):
            s = pl.ds(g * _L, _L)
            colu = (lax.shift_right_logical(ui_v[c, s], 11) & 3) * DIM
            colp = (lax.shift_right_logical(pi_v[c, s], 11) & 3) * DIM
            coln = (lax.shift_right_logical(ni_v[c, s], 11) & 3) * DIM
            rowv = g * _L + lanes
            acc = jnp.zeros((_L,), jnp.float32)
            for d in range(DIM):
                u = plsc.load_gather(ub, [rowv, colu + d])
                p = plsc.load_gather(pb, [rowv, colp + d])
                n = plsc.load_gather(nb, [rowv, coln + d])
                acc = acc + u * (p - n)
            out_v[pl.ds(c * _CHUNK + g * _L, _L)] = acc
            return 0

        lax.fori_loop(0, _CHUNK // _L, group, 0)

    pltpu.sync_copy(out_v, out_hbm.at[pl.ds(base, _BPW)])


def kernel(ui, pi, ni, gamma_users, gamma_items):
    n_users = gamma_users.shape[0]
    q_pad = -(-(n_users // PACK) // _TROWS) * _TROWS
    gu2 = _transpose_pack(gamma_users.T, q_pad)
    gi2 = _transpose_pack(gamma_items.T, q_pad)
    ui3 = ui.astype(jnp.int32).reshape(_NW, _NCHUNK, _CHUNK)
    pi3 = pi.astype(jnp.int32).reshape(_NW, _NCHUNK, _CHUNK)
    ni3 = ni.astype(jnp.int32).reshape(_NW, _NCHUNK, _CHUNK)
    return _bpr_sc(ui3, pi3, ni3, gu2, gi2)


# speedup vs baseline: 1.6351x; 1.0243x over previous
import functools

import jax
import jax.numpy as jnp
from jax import lax
from jax.experimental import pallas as pl
from jax.experimental.pallas import tpu as pltpu
from jax.experimental.pallas import tpu_sc as plsc

DIM = 32
BATCH = 16384
PACK = 128 // DIM

_INFO = plsc.get_sparse_core_info()
_NC, _NS, _L = _INFO.num_cores, _INFO.num_subcores, _INFO.num_lanes
_NW = _NC * _NS
_BPW = BATCH // _NW
_CHUNK = 128
_NCHUNK = _BPW // _CHUNK

_TROWS = 8192


def _tpose_body(x_ref, out_ref):
    eye = jnp.eye(DIM, dtype=jnp.float32)
    xt = lax.dot_general(x_ref[...], eye, (((0,), (0,)), ((), ())),
                         preferred_element_type=jnp.float32)
    out_ref[...] = jnp.concatenate(
        [xt[q * _TROWS:(q + 1) * _TROWS] for q in range(PACK)], axis=1)


def _transpose_pack(gT, q_pad):
    dim, n = gT.shape
    grid = (q_pad // _TROWS,)
    return pl.pallas_call(
        _tpose_body,
        grid=grid,
        in_specs=[pl.BlockSpec((dim, PACK * _TROWS), lambda i: (0, i))],
        out_specs=pl.BlockSpec((_TROWS, PACK * dim), lambda i: (i, 0)),
        out_shape=jax.ShapeDtypeStruct((q_pad, PACK * dim), jnp.float32),
    )(gT)


_mesh = plsc.VectorSubcoreMesh(core_axis_name="c", subcore_axis_name="s")


@functools.partial(
    pl.kernel,
    mesh=_mesh,
    out_type=jax.ShapeDtypeStruct((BATCH,), jnp.float32),
    compiler_params=pltpu.CompilerParams(
        needs_layout_passes=False, use_tc_tiling_on_sc=True),
    scratch_types=[
        pltpu.VMEM((_NCHUNK, _CHUNK), jnp.int32),
        pltpu.VMEM((_NCHUNK, _CHUNK), jnp.int32),
        pltpu.VMEM((_NCHUNK, _CHUNK), jnp.int32),
        pltpu.VMEM((_NCHUNK, _CHUNK), jnp.int32),
        pltpu.VMEM((_NCHUNK, _CHUNK), jnp.int32),
        pltpu.VMEM((_NCHUNK, _CHUNK), jnp.int32),
        pltpu.VMEM((2, _CHUNK, 128), jnp.float32),
        pltpu.VMEM((2, _CHUNK, 128), jnp.float32),
        pltpu.VMEM((2, _CHUNK, 128), jnp.float32),
        pltpu.VMEM((_BPW,), jnp.float32),
        pltpu.SemaphoreType.DMA,
        pltpu.SemaphoreType.DMA,
    ],
)
def _bpr_sc(ui_hbm, pi_hbm, ni_hbm, gu_hbm, gi_hbm, out_hbm,
            ui_v, pi_v, ni_v, upk_v, ppk_v, npk_v,
            u_rows, p_rows, n_rows, out_v, sem0, sem1):
    wid = lax.axis_index("s") * _NC + lax.axis_index("c")
    base = wid * _BPW
    sems = (sem0, sem1)

    pltpu.sync_copy(ui_hbm.at[wid], ui_v)
    pltpu.sync_copy(pi_hbm.at[wid], pi_v)
    pltpu.sync_copy(ni_hbm.at[wid], ni_v)

    def _pk(v):
        return lax.shift_left(lax.shift_right_logical(v, 15), 13) | (v & 8191)

    for c in range(_NCHUNK):
        for g in range(_CHUNK // _L):
            s = pl.ds(g * _L, _L)
            upk_v[c, s] = _pk(ui_v[c, s])
            ppk_v[c, s] = _pk(pi_v[c, s])
            npk_v[c, s] = _pk(ni_v[c, s])

    def fire(c):
        b = c % 2
        sem = sems[b]
        return (pltpu.async_copy(gu_hbm.at[upk_v.at[c]], u_rows.at[b], sem),
                pltpu.async_copy(gi_hbm.at[ppk_v.at[c]], p_rows.at[b], sem),
                pltpu.async_copy(gi_hbm.at[npk_v.at[c]], n_rows.at[b], sem))

    lanes = lax.iota(jnp.int32, _L)
    inflight = fire(0)

    for c in range(_NCHUNK):
        b = c % 2
        cur = inflight
        if c + 1 < _NCHUNK:
            inflight = fire(c + 1)
        for cp in cur:
            cp.wait()

        ub, pb, nb = u_rows.at[b], p_rows.at[b], n_rows.at[b]

        def group(g, _, c=c, ub=ub, pb=pb, nb=nb):
            s = pl.ds(g * _L, _L)
            colu = (lax.shift_right_logical(ui_v[c, s], 13) & 3) * DIM
            colp = (lax.shift_right_logical(pi_v[c, s], 13) & 3) * DIM
            coln = (lax.shift_right_logical(ni_v[c, s], 13) & 3) * DIM
            rowv = g * _L + lanes
            acc = jnp.zeros((_L,), jnp.float32)
            for d in range(DIM):
                u = plsc.load_gather(ub, [rowv, colu + d])
                p = plsc.load_gather(pb, [rowv, colp + d])
                n = plsc.load_gather(nb, [rowv, coln + d])
                acc = acc + u * (p - n)
            out_v[pl.ds(c * _CHUNK + g * _L, _L)] = acc
            return 0

        lax.fori_loop(0, _CHUNK // _L, group, 0)

    pltpu.sync_copy(out_v, out_hbm.at[pl.ds(base, _BPW)])


def kernel(ui, pi, ni, gamma_users, gamma_items):
    n_users = gamma_users.shape[0]
    q_pad = -(-(n_users // PACK) // _TROWS) * _TROWS
    gu2 = _transpose_pack(gamma_users.T, q_pad)
    gi2 = _transpose_pack(gamma_items.T, q_pad)
    ui3 = ui.astype(jnp.int32).reshape(_NW, _NCHUNK, _CHUNK)
    pi3 = pi.astype(jnp.int32).reshape(_NW, _NCHUNK, _CHUNK)
    ni3 = ni.astype(jnp.int32).reshape(_NW, _NCHUNK, _CHUNK)
    return _bpr_sc(ui3, pi3, ni3, gu2, gi2)


# speedup vs baseline: 2.4675x; 1.5091x over previous
import functools

import jax
import jax.numpy as jnp
from jax import lax
from jax.experimental import pallas as pl
from jax.experimental.pallas import tpu as pltpu
from jax.experimental.pallas import tpu_sc as plsc

DIM = 32
BATCH = 16384
PACK = 128 // DIM

_INFO = plsc.get_sparse_core_info()
_NC, _NS, _L = _INFO.num_cores, _INFO.num_subcores, _INFO.num_lanes
_NW = _NC * _NS
_BPW = BATCH // _NW
_CHUNK = 128
_NCHUNK = _BPW // _CHUNK

_TROWS = 4096


def _tpose_body(x_ref, out_ref):
    rows = lax.broadcasted_iota(jnp.int32, (DIM, PACK * DIM), 0)
    cols = lax.broadcasted_iota(jnp.int32, (DIM, PACK * DIM), 1)
    acc = None
    for q in range(PACK):
        sel = (cols == rows + q * DIM).astype(jnp.float32)
        xq = x_ref[:, q * _TROWS:(q + 1) * _TROWS]
        t = lax.dot_general(xq, sel, (((0,), (0,)), ((), ())),
                            preferred_element_type=jnp.float32)
        acc = t if acc is None else acc + t
    out_ref[...] = acc


def _transpose_pack(gT, q_pad):
    dim, n = gT.shape
    grid = (q_pad // _TROWS,)
    return pl.pallas_call(
        _tpose_body,
        grid=grid,
        in_specs=[pl.BlockSpec((dim, PACK * _TROWS), lambda i: (0, i))],
        out_specs=pl.BlockSpec((_TROWS, PACK * dim), lambda i: (i, 0)),
        out_shape=jax.ShapeDtypeStruct((q_pad, PACK * dim), jnp.float32),
    )(gT)


_mesh = plsc.VectorSubcoreMesh(core_axis_name="c", subcore_axis_name="s")


@functools.partial(
    pl.kernel,
    mesh=_mesh,
    out_type=jax.ShapeDtypeStruct((BATCH,), jnp.float32),
    compiler_params=pltpu.CompilerParams(
        needs_layout_passes=False, use_tc_tiling_on_sc=True),
    scratch_types=[
        pltpu.VMEM((_NCHUNK, _CHUNK), jnp.int32),
        pltpu.VMEM((_NCHUNK, _CHUNK), jnp.int32),
        pltpu.VMEM((_NCHUNK, _CHUNK), jnp.int32),
        pltpu.VMEM((_NCHUNK, _CHUNK), jnp.int32),
        pltpu.VMEM((_NCHUNK, _CHUNK), jnp.int32),
        pltpu.VMEM((_NCHUNK, _CHUNK), jnp.int32),
        pltpu.VMEM((2, _CHUNK, 128), jnp.float32),
        pltpu.VMEM((2, _CHUNK, 128), jnp.float32),
        pltpu.VMEM((2, _CHUNK, 128), jnp.float32),
        pltpu.VMEM((_BPW,), jnp.float32),
        pltpu.SemaphoreType.DMA,
        pltpu.SemaphoreType.DMA,
    ],
)
def _bpr_sc(ui_hbm, pi_hbm, ni_hbm, gu_hbm, gi_hbm, out_hbm,
            ui_v, pi_v, ni_v, upk_v, ppk_v, npk_v,
            u_rows, p_rows, n_rows, out_v, sem0, sem1):
    wid = lax.axis_index("s") * _NC + lax.axis_index("c")
    base = wid * _BPW
    sems = (sem0, sem1)

    pltpu.sync_copy(ui_hbm.at[wid], ui_v)
    pltpu.sync_copy(pi_hbm.at[wid], pi_v)
    pltpu.sync_copy(ni_hbm.at[wid], ni_v)

    def _pk(v):
        return lax.shift_left(lax.shift_right_logical(v, 14), 12) | (v & 4095)

    for c in range(_NCHUNK):
        for g in range(_CHUNK // _L):
            s = pl.ds(g * _L, _L)
            upk_v[c, s] = _pk(ui_v[c, s])
            ppk_v[c, s] = _pk(pi_v[c, s])
            npk_v[c, s] = _pk(ni_v[c, s])

    def fire(c):
        b = c % 2
        sem = sems[b]
        return (pltpu.async_copy(gu_hbm.at[upk_v.at[c]], u_rows.at[b], sem),
                pltpu.async_copy(gi_hbm.at[ppk_v.at[c]], p_rows.at[b], sem),
                pltpu.async_copy(gi_hbm.at[npk_v.at[c]], n_rows.at[b], sem))

    lanes = lax.iota(jnp.int32, _L)
    inflight = fire(0)

    for c in range(_NCHUNK):
        b = c % 2
        cur = inflight
        if c + 1 < _NCHUNK:
            inflight = fire(c + 1)
        for cp in cur:
            cp.wait()

        ub, pb, nb = u_rows.at[b], p_rows.at[b], n_rows.at[b]

        def group(g, _, c=c, ub=ub, pb=pb, nb=nb):
            s = pl.ds(g * _L, _L)
            colu = (lax.shift_right_logical(ui_v[c, s], 12) & 3) * DIM
            colp = (lax.shift_right_logical(pi_v[c, s], 12) & 3) * DIM
            coln = (lax.shift_right_logical(ni_v[c, s], 12) & 3) * DIM
            rowv = g * _L + lanes
            acc = jnp.zeros((_L,), jnp.float32)
            for d in range(DIM):
                u = plsc.load_gather(ub, [rowv, colu + d])
                p = plsc.load_gather(pb, [rowv, colp + d])
                n = plsc.load_gather(nb, [rowv, coln + d])
                acc = acc + u * (p - n)
            out_v[pl.ds(c * _CHUNK + g * _L, _L)] = acc
            return 0

        lax.fori_loop(0, _CHUNK // _L, group, 0)

    pltpu.sync_copy(out_v, out_hbm.at[pl.ds(base, _BPW)])


def kernel(ui, pi, ni, gamma_users, gamma_items):
    n_users = gamma_users.shape[0]
    q_pad = -(-(n_users // PACK) // _TROWS) * _TROWS
    gu2 = _transpose_pack(gamma_users.T, q_pad)
    gi2 = _transpose_pack(gamma_items.T, q_pad)
    ui3 = ui.astype(jnp.int32).reshape(_NW, _NCHUNK, _CHUNK)
    pi3 = pi.astype(jnp.int32).reshape(_NW, _NCHUNK, _CHUNK)
    ni3 = ni.astype(jnp.int32).reshape(_NW, _NCHUNK, _CHUNK)
    return _bpr_sc(ui3, pi3, ni3, gu2, gi2)


# speedup vs baseline: 3.5698x; 1.4467x over previous
import functools

import jax
import jax.numpy as jnp
from jax import lax
from jax.experimental import pallas as pl
from jax.experimental.pallas import tpu as pltpu
from jax.experimental.pallas import tpu_sc as plsc

DIM = 32
BATCH = 16384
PACK = 128 // DIM

_INFO = plsc.get_sparse_core_info()
_NC, _NS, _L = _INFO.num_cores, _INFO.num_subcores, _INFO.num_lanes
_NW = _NC * _NS
_BPW = BATCH // _NW
_CHUNK = 128
_NCHUNK = _BPW // _CHUNK

_TROWS = 4096


def _tpose_body(x_ref, out_ref):
    xc = jnp.concatenate(
        [x_ref[:, q * _TROWS:(q + 1) * _TROWS] for q in range(PACK)], axis=0)
    eye = jnp.eye(PACK * DIM, dtype=jnp.float32)
    out_ref[...] = lax.dot_general(xc, eye, (((0,), (0,)), ((), ())),
                                   preferred_element_type=jnp.float32)


def _transpose_pack(gT, q_pad):
    dim, n = gT.shape
    grid = (q_pad // _TROWS,)
    return pl.pallas_call(
        _tpose_body,
        grid=grid,
        in_specs=[pl.BlockSpec((dim, PACK * _TROWS), lambda i: (0, i))],
        out_specs=pl.BlockSpec((_TROWS, PACK * dim), lambda i: (i, 0)),
        out_shape=jax.ShapeDtypeStruct((q_pad, PACK * dim), jnp.float32),
    )(gT)


_mesh = plsc.VectorSubcoreMesh(core_axis_name="c", subcore_axis_name="s")


@functools.partial(
    pl.kernel,
    mesh=_mesh,
    out_type=jax.ShapeDtypeStruct((BATCH,), jnp.float32),
    compiler_params=pltpu.CompilerParams(
        needs_layout_passes=False, use_tc_tiling_on_sc=True),
    scratch_types=[
        pltpu.VMEM((_NCHUNK, _CHUNK), jnp.int32),
        pltpu.VMEM((_NCHUNK, _CHUNK), jnp.int32),
        pltpu.VMEM((_NCHUNK, _CHUNK), jnp.int32),
        pltpu.VMEM((_NCHUNK, _CHUNK), jnp.int32),
        pltpu.VMEM((_NCHUNK, _CHUNK), jnp.int32),
        pltpu.VMEM((_NCHUNK, _CHUNK), jnp.int32),
        pltpu.VMEM((2, _CHUNK, 128), jnp.float32),
        pltpu.VMEM((2, _CHUNK, 128), jnp.float32),
        pltpu.VMEM((2, _CHUNK, 128), jnp.float32),
        pltpu.VMEM((_BPW,), jnp.float32),
        pltpu.SemaphoreType.DMA,
        pltpu.SemaphoreType.DMA,
    ],
)
def _bpr_sc(ui_hbm, pi_hbm, ni_hbm, gu_hbm, gi_hbm, out_hbm,
            ui_v, pi_v, ni_v, upk_v, ppk_v, npk_v,
            u_rows, p_rows, n_rows, out_v, sem0, sem1):
    wid = lax.axis_index("s") * _NC + lax.axis_index("c")
    base = wid * _BPW
    sems = (sem0, sem1)

    pltpu.sync_copy(ui_hbm.at[wid], ui_v)
    pltpu.sync_copy(pi_hbm.at[wid], pi_v)
    pltpu.sync_copy(ni_hbm.at[wid], ni_v)

    def _pk(v):
        return lax.shift_left(lax.shift_right_logical(v, 14), 12) | (v & 4095)

    for c in range(_NCHUNK):
        for g in range(_CHUNK // _L):
            s = pl.ds(g * _L, _L)
            upk_v[c, s] = _pk(ui_v[c, s])
            ppk_v[c, s] = _pk(pi_v[c, s])
            npk_v[c, s] = _pk(ni_v[c, s])

    def fire(c):
        b = c % 2
        sem = sems[b]
        return (pltpu.async_copy(gu_hbm.at[upk_v.at[c]], u_rows.at[b], sem),
                pltpu.async_copy(gi_hbm.at[ppk_v.at[c]], p_rows.at[b], sem),
                pltpu.async_copy(gi_hbm.at[npk_v.at[c]], n_rows.at[b], sem))

    lanes = lax.iota(jnp.int32, _L)
    inflight = fire(0)

    for c in range(_NCHUNK):
        b = c % 2
        cur = inflight
        if c + 1 < _NCHUNK:
            inflight = fire(c + 1)
        for cp in cur:
            cp.wait()

        ub, pb, nb = u_rows.at[b], p_rows.at[b], n_rows.at[b]

        def group(g, _, c=c, ub=ub, pb=pb, nb=nb):
            s = pl.ds(g * _L, _L)
            colu = (lax.shift_right_logical(ui_v[c, s], 12) & 3) * DIM
            colp = (lax.shift_right_logical(pi_v[c, s], 12) & 3) * DIM
            coln = (lax.shift_right_logical(ni_v[c, s], 12) & 3) * DIM
            rowv = g * _L + lanes
            acc = jnp.zeros((_L,), jnp.float32)
            for d in range(DIM):
                u = plsc.load_gather(ub, [rowv, colu + d])
                p = plsc.load_gather(pb, [rowv, colp + d])
                n = plsc.load_gather(nb, [rowv, coln + d])
                acc = acc + u * (p - n)
            out_v[pl.ds(c * _CHUNK + g * _L, _L)] = acc
            return 0

        lax.fori_loop(0, _CHUNK // _L, group, 0)

    pltpu.sync_copy(out_v, out_hbm.at[pl.ds(base, _BPW)])


def kernel(ui, pi, ni, gamma_users, gamma_items):
    n_users = gamma_users.shape[0]
    q_pad = -(-(n_users // PACK) // _TROWS) * _TROWS
    gu2 = _transpose_pack(gamma_users.T, q_pad)
    gi2 = _transpose_pack(gamma_items.T, q_pad)
    ui3 = ui.astype(jnp.int32).reshape(_NW, _NCHUNK, _CHUNK)
    pi3 = pi.astype(jnp.int32).reshape(_NW, _NCHUNK, _CHUNK)
    ni3 = ni.astype(jnp.int32).reshape(_NW, _NCHUNK, _CHUNK)
    return _bpr_sc(ui3, pi3, ni3, gu2, gi2)


# speedup vs baseline: 4.0782x; 1.1424x over previous
import functools

import jax
import jax.numpy as jnp
from jax import lax
from jax.experimental import pallas as pl
from jax.experimental.pallas import tpu as pltpu
from jax.experimental.pallas import tpu_sc as plsc

DIM = 32
BATCH = 16384
PACK = 128 // DIM

_INFO = plsc.get_sparse_core_info()
_NC, _NS, _L = _INFO.num_cores, _INFO.num_subcores, _INFO.num_lanes
_NW = _NC * _NS
_BPW = BATCH // _NW
_CHUNK = 128
_NCHUNK = _BPW // _CHUNK

_TROWS = 8192


def _tpose_body(x_ref, out_ref):
    xc = jnp.concatenate(
        [x_ref[:, q * _TROWS:(q + 1) * _TROWS] for q in range(PACK)], axis=0)
    eye = jnp.eye(PACK * DIM, dtype=jnp.float32)
    out_ref[...] = lax.dot_general(xc, eye, (((0,), (0,)), ((), ())),
                                   preferred_element_type=jnp.float32)


def _transpose_pack(gT, q_pad):
    dim, n = gT.shape
    grid = (q_pad // _TROWS,)
    return pl.pallas_call(
        _tpose_body,
        grid=grid,
        in_specs=[pl.BlockSpec((dim, PACK * _TROWS), lambda i: (0, i))],
        out_specs=pl.BlockSpec((_TROWS, PACK * dim), lambda i: (i, 0)),
        out_shape=jax.ShapeDtypeStruct((q_pad, PACK * dim), jnp.float32),
    )(gT)


_mesh = plsc.VectorSubcoreMesh(core_axis_name="c", subcore_axis_name="s")


@functools.partial(
    pl.kernel,
    mesh=_mesh,
    out_type=jax.ShapeDtypeStruct((BATCH,), jnp.float32),
    compiler_params=pltpu.CompilerParams(
        needs_layout_passes=False, use_tc_tiling_on_sc=True),
    scratch_types=[
        pltpu.VMEM((_NCHUNK, _CHUNK), jnp.int32),
        pltpu.VMEM((_NCHUNK, _CHUNK), jnp.int32),
        pltpu.VMEM((_NCHUNK, _CHUNK), jnp.int32),
        pltpu.VMEM((_NCHUNK, _CHUNK), jnp.int32),
        pltpu.VMEM((_NCHUNK, _CHUNK), jnp.int32),
        pltpu.VMEM((_NCHUNK, _CHUNK), jnp.int32),
        pltpu.VMEM((2, _CHUNK, 128), jnp.float32),
        pltpu.VMEM((2, _CHUNK, 128), jnp.float32),
        pltpu.VMEM((2, _CHUNK, 128), jnp.float32),
        pltpu.VMEM((_BPW,), jnp.float32),
        pltpu.SemaphoreType.DMA,
        pltpu.SemaphoreType.DMA,
    ],
)
def _bpr_sc(ui_hbm, pi_hbm, ni_hbm, gu_hbm, gi_hbm, out_hbm,
            ui_v, pi_v, ni_v, upk_v, ppk_v, npk_v,
            u_rows, p_rows, n_rows, out_v, sem0, sem1):
    wid = lax.axis_index("s") * _NC + lax.axis_index("c")
    base = wid * _BPW
    sems = (sem0, sem1)

    pltpu.sync_copy(ui_hbm.at[wid], ui_v)
    pltpu.sync_copy(pi_hbm.at[wid], pi_v)
    pltpu.sync_copy(ni_hbm.at[wid], ni_v)

    def _pk(v):
        return lax.shift_left(lax.shift_right_logical(v, 15), 13) | (v & 8191)

    for c in range(_NCHUNK):
        for g in range(_CHUNK // _L):
            s = pl.ds(g * _L, _L)
            upk_v[c, s] = _pk(ui_v[c, s])
            ppk_v[c, s] = _pk(pi_v[c, s])
            npk_v[c, s] = _pk(ni_v[c, s])

    def fire(c):
        b = c % 2
        sem = sems[b]
        return (pltpu.async_copy(gu_hbm.at[upk_v.at[c]], u_rows.at[b], sem),
                pltpu.async_copy(gi_hbm.at[ppk_v.at[c]], p_rows.at[b], sem),
                pltpu.async_copy(gi_hbm.at[npk_v.at[c]], n_rows.at[b], sem))

    lanes = lax.iota(jnp.int32, _L)
    inflight = fire(0)

    for c in range(_NCHUNK):
        b = c % 2
        cur = inflight
        if c + 1 < _NCHUNK:
            inflight = fire(c + 1)
        for cp in cur:
            cp.wait()

        ub, pb, nb = u_rows.at[b], p_rows.at[b], n_rows.at[b]

        def group(g, _, c=c, ub=ub, pb=pb, nb=nb):
            s = pl.ds(g * _L, _L)
            colu = (lax.shift_right_logical(ui_v[c, s], 13) & 3) * DIM
            colp = (lax.shift_right_logical(pi_v[c, s], 13) & 3) * DIM
            coln = (lax.shift_right_logical(ni_v[c, s], 13) & 3) * DIM
            rowv = g * _L + lanes
            acc = jnp.zeros((_L,), jnp.float32)
            for d in range(DIM):
                u = plsc.load_gather(ub, [rowv, colu + d])
                p = plsc.load_gather(pb, [rowv, colp + d])
                n = plsc.load_gather(nb, [rowv, coln + d])
                acc = acc + u * (p - n)
            out_v[pl.ds(c * _CHUNK + g * _L, _L)] = acc
            return 0

        lax.fori_loop(0, _CHUNK // _L, group, 0)

    pltpu.sync_copy(out_v, out_hbm.at[pl.ds(base, _BPW)])


def kernel(ui, pi, ni, gamma_users, gamma_items):
    n_users = gamma_users.shape[0]
    q_pad = -(-(n_users // PACK) // _TROWS) * _TROWS
    gu2 = _transpose_pack(gamma_users.T, q_pad)
    gi2 = _transpose_pack(gamma_items.T, q_pad)
    ui3 = ui.astype(jnp.int32).reshape(_NW, _NCHUNK, _CHUNK)
    pi3 = pi.astype(jnp.int32).reshape(_NW, _NCHUNK, _CHUNK)
    ni3 = ni.astype(jnp.int32).reshape(_NW, _NCHUNK, _CHUNK)
    return _bpr_sc(ui3, pi3, ni3, gu2, gi2)
